# Initial kernel scaffold; baseline (speedup 1.0000x reference)
import jax
import jax.numpy as jnp
from jax.experimental import pallas as pl


def kernel(x, edge_index, edge_weight, W, b):
    raise NotImplementedError("write your pallas kernel here")


# speedup vs baseline: 10.6275x; 10.6275x over previous
import functools

import jax
import jax.numpy as jnp
from jax import lax
from jax.experimental import pallas as pl
from jax.experimental.pallas import tpu as pltpu
from jax.experimental.pallas import tpu_sc as plsc

N_CORES = 2
N_SUBCORES = 16
NW = N_CORES * N_SUBCORES
LANES = 16
K = 128
BR = 256


def _sc_mesh():
    return plsc.VectorSubcoreMesh(core_axis_name="c", subcore_axis_name="s")


def _make_deg_kernel(e_pad, n_pad):
    cpt = e_pad // (NW * K)
    rows_per_tile = n_pad // N_SUBCORES

    @functools.partial(
        pl.kernel,
        out_type=jax.ShapeDtypeStruct((N_CORES, n_pad), jnp.float32),
        mesh=_sc_mesh(),
        compiler_params=pltpu.CompilerParams(needs_layout_passes=False),
        scratch_types=[
            pltpu.VMEM((K,), jnp.int32),
            pltpu.VMEM((K,), jnp.float32),
            pltpu.VMEM((rows_per_tile,), jnp.float32),
            pltpu.VMEM_SHARED((n_pad,), jnp.float32),
            pltpu.SemaphoreType.DMA,
        ],
    )
    def deg_kernel(col_hbm, ew_hbm, deg_hbm, colbuf, ewbuf, zbuf, acc, sem):
        c = lax.axis_index("c")
        s = lax.axis_index("s")
        wid = c * N_SUBCORES + s

        @pl.loop(0, rows_per_tile // LANES)
        def _zero(i):
            zbuf[pl.ds(i * LANES, LANES)] = jnp.zeros((LANES,), jnp.float32)

        pltpu.sync_copy(zbuf, acc.at[pl.ds(s * rows_per_tile, rows_per_tile)])
        plsc.subcore_barrier()

        @pl.loop(0, cpt)
        def _chunk(g):
            base = (wid * cpt + g) * K
            pltpu.sync_copy(col_hbm.at[pl.ds(base, K)], colbuf)
            pltpu.sync_copy(ew_hbm.at[pl.ds(base, K)], ewbuf)
            pltpu.async_copy(ewbuf, acc.at[colbuf], sem, add=True).wait()

        plsc.subcore_barrier()
        pltpu.sync_copy(
            acc.at[pl.ds(s * rows_per_tile, rows_per_tile)],
            deg_hbm.at[c, pl.ds(s * rows_per_tile, rows_per_tile)],
        )

    return deg_kernel


def _make_agg_kernel(e_pad, n_pad, d):
    cpt = e_pad // (NW * K)
    rows_per_tile = n_pad // N_SUBCORES
    groups = d // LANES

    @functools.partial(
        pl.kernel,
        out_type=jax.ShapeDtypeStruct((N_CORES, n_pad, d), jnp.float32),
        mesh=_sc_mesh(),
        compiler_params=pltpu.CompilerParams(needs_layout_passes=False),
        scratch_types=[
            pltpu.VMEM((K,), jnp.int32),
            pltpu.VMEM((K,), jnp.int32),
            pltpu.VMEM((K,), jnp.float32),
            pltpu.VMEM((K, d), jnp.float32),
            pltpu.VMEM_SHARED((n_pad, d), jnp.float32),
            pltpu.SemaphoreType.DMA,
            pltpu.SemaphoreType.DMA,
        ],
    )
    def agg_kernel(row_hbm, col_hbm, ew_hbm, y_hbm, agg_hbm,
                   rowbuf, colbuf, ewbuf, rows, acc, gsem, ssem):
        c = lax.axis_index("c")
        s = lax.axis_index("s")
        wid = c * N_SUBCORES + s

        @pl.loop(0, K)
        def _zero(i):
            for f in range(groups):
                rows[i, pl.ds(f * LANES, LANES)] = jnp.zeros((LANES,), jnp.float32)

        @pl.loop(0, rows_per_tile // K)
        def _zacc(t):
            pltpu.sync_copy(rows, acc.at[pl.ds(s * rows_per_tile + t * K, K)])

        plsc.subcore_barrier()

        @pl.loop(0, cpt)
        def _chunk(g):
            base = (wid * cpt + g) * K
            pltpu.sync_copy(row_hbm.at[pl.ds(base, K)], rowbuf)
            pltpu.sync_copy(col_hbm.at[pl.ds(base, K)], colbuf)
            pltpu.sync_copy(ew_hbm.at[pl.ds(base, K)], ewbuf)
            pltpu.async_copy(y_hbm.at[rowbuf], rows, gsem).wait()

            @pl.loop(0, K)
            def _scale(j):
                jv = jnp.broadcast_to(j, (LANES,)).astype(jnp.int32)
                sv = plsc.load_gather(ewbuf, [jv])
                for f in range(groups):
                    rows[j, pl.ds(f * LANES, LANES)] = (
                        rows[j, pl.ds(f * LANES, LANES)] * sv
                    )

            pltpu.async_copy(rows, acc.at[colbuf], ssem, add=True).wait()

        plsc.subcore_barrier()
        pltpu.sync_copy(
            acc.at[pl.ds(s * rows_per_tile, rows_per_tile)],
            agg_hbm.at[c, pl.ds(s * rows_per_tile, rows_per_tile)],
        )

    return agg_kernel


def _tc_xw_dis(x_p, W, deg3, n_pad, d):
    nb = n_pad // BR

    def body(xb, wb, degb, yb, disb):
        dcol = degb[0] + degb[1] + 1.0
        dis = jnp.where(dcol > 0, lax.rsqrt(dcol), 0.0)
        xw = jnp.dot(xb[...], wb[...], preferred_element_type=jnp.float32)
        yb[...] = xw * dis
        disb[...] = dis

    return pl.pallas_call(
        body,
        grid=(nb,),
        in_specs=[
            pl.BlockSpec((BR, d), lambda i: (i, 0)),
            pl.BlockSpec((d, d), lambda i: (0, 0)),
            pl.BlockSpec((N_CORES, BR, 1), lambda i: (0, i, 0)),
        ],
        out_specs=[
            pl.BlockSpec((BR, d), lambda i: (i, 0)),
            pl.BlockSpec((BR, 1), lambda i: (i, 0)),
        ],
        out_shape=[
            jax.ShapeDtypeStruct((n_pad, d), jnp.float32),
            jax.ShapeDtypeStruct((n_pad, 1), jnp.float32),
        ],
    )(x_p, W, deg3)


def _tc_final(agg, y, dis, b2, n_pad, d):
    nb = n_pad // BR

    def body(aggb, yb, disb, bb, ob):
        s = (aggb[0] + aggb[1] + yb[...]) * disb[...] + bb[...]
        ob[...] = jnp.maximum(s, 0.0)

    return pl.pallas_call(
        body,
        grid=(nb,),
        in_specs=[
            pl.BlockSpec((N_CORES, BR, d), lambda i: (0, i, 0)),
            pl.BlockSpec((BR, d), lambda i: (i, 0)),
            pl.BlockSpec((BR, 1), lambda i: (i, 0)),
            pl.BlockSpec((1, d), lambda i: (0, 0)),
        ],
        out_specs=pl.BlockSpec((BR, d), lambda i: (i, 0)),
        out_shape=jax.ShapeDtypeStruct((n_pad, d), jnp.float32),
    )(agg, y, dis, b2)


def kernel(x, edge_index, edge_weight, W, b):
    n, d = x.shape
    e = edge_index.shape[1]

    n_pad = ((n + NW * LANES - 1) // (NW * LANES)) * (NW * LANES)
    e_pad = ((e + NW * K - 1) // (NW * K)) * (NW * K)

    row = edge_index[0].astype(jnp.int32)
    col = edge_index[1].astype(jnp.int32)
    ew = edge_weight.astype(jnp.float32)
    if e_pad != e:
        pad = e_pad - e
        row = jnp.concatenate([row, jnp.zeros((pad,), jnp.int32)])
        col = jnp.concatenate([col, jnp.zeros((pad,), jnp.int32)])
        ew = jnp.concatenate([ew, jnp.zeros((pad,), jnp.float32)])
    x_p = x
    if n_pad != n:
        x_p = jnp.concatenate(
            [x, jnp.zeros((n_pad - n, d), jnp.float32)], axis=0)

    deg_p = _make_deg_kernel(e_pad, n_pad)(col, ew)
    deg3 = deg_p.reshape(N_CORES, n_pad, 1)
    y, dis = _tc_xw_dis(x_p, W, deg3, n_pad, d)
    agg = _make_agg_kernel(e_pad, n_pad, d)(row, col, ew, y)
    out = _tc_final(agg, y, dis, b.reshape(1, d), n_pad, d)
    return out[:n]


# speedup vs baseline: 12.8015x; 1.2046x over previous
import functools

import jax
import jax.numpy as jnp
from jax import lax
from jax.experimental import pallas as pl
from jax.experimental.pallas import tpu as pltpu
from jax.experimental.pallas import tpu_sc as plsc

N_CORES = 2
N_SUBCORES = 16
NW = N_CORES * N_SUBCORES
LANES = 16
K = 64
NBUF = 4
NIDX = 8
BR = 256


def _sc_mesh():
    return plsc.VectorSubcoreMesh(core_axis_name="c", subcore_axis_name="s")


def _make_deg_kernel(e_pad, n_pad):
    cpt = e_pad // (NW * K)
    rows_per_tile = n_pad // N_SUBCORES

    @functools.partial(
        pl.kernel,
        out_type=jax.ShapeDtypeStruct((N_CORES, n_pad), jnp.float32),
        mesh=_sc_mesh(),
        compiler_params=pltpu.CompilerParams(needs_layout_passes=False),
        scratch_types=(
            [pltpu.VMEM((K,), jnp.int32) for _ in range(NBUF)]
            + [pltpu.VMEM((K,), jnp.float32) for _ in range(NBUF)]
            + [pltpu.VMEM((rows_per_tile,), jnp.float32)]
            + [pltpu.SemaphoreType.DMA for _ in range(2 * NBUF)]
            + [pltpu.VMEM_SHARED((n_pad,), jnp.float32)]
        ),
    )
    def deg_kernel(col_hbm, ew_hbm, deg_hbm, *refs):
        colb = refs[0:NBUF]
        ewb = refs[NBUF:2 * NBUF]
        zbuf = refs[2 * NBUF]
        isem = refs[2 * NBUF + 1:2 * NBUF + 1 + NBUF]
        ssem = refs[2 * NBUF + 1 + NBUF:2 * NBUF + 1 + 2 * NBUF]
        acc = refs[2 * NBUF + 1 + 2 * NBUF]

        c_ax = lax.axis_index("c")
        s_ax = lax.axis_index("s")
        wid = c_ax * N_SUBCORES + s_ax
        tile_base = wid * cpt * K

        @pl.loop(0, rows_per_tile // LANES)
        def _zero(i):
            zbuf[pl.ds(i * LANES, LANES)] = jnp.zeros((LANES,), jnp.float32)

        pltpu.sync_copy(zbuf, acc.at[pl.ds(s_ax * rows_per_tile, rows_per_tile)])
        plsc.subcore_barrier()

        def issue_idx(ch, b):
            base = tile_base + ch * K
            pltpu.async_copy(col_hbm.at[pl.ds(base, K)], colb[b], isem[b])
            pltpu.async_copy(ew_hbm.at[pl.ds(base, K)], ewb[b], isem[b])

        def wait_idx(b):
            pltpu.make_async_copy(col_hbm.at[pl.ds(0, K)], colb[b], isem[b]).wait()
            pltpu.make_async_copy(ew_hbm.at[pl.ds(0, K)], ewb[b], isem[b]).wait()

        def wait_scat(b):
            pltpu.make_async_copy(ewb[b], acc.at[colb[b]], ssem[b]).wait()

        issue_idx(0, 0)
        issue_idx(1, 1)

        @pl.loop(0, cpt, step=NBUF)
        def _main(g):
            for b in range(NBUF):
                ch = g + b
                t = (b + 2) % NBUF

                @pl.when(ch + 2 < cpt)
                def _prep():
                    @pl.when(ch >= 2)
                    def _drain():
                        wait_scat(t)
                    issue_idx(ch + 2, t)

                wait_idx(b)
                pltpu.async_copy(ewb[b], acc.at[colb[b]], ssem[b], add=True)

        for b in range(NBUF):
            wait_scat(b)

        plsc.subcore_barrier()
        pltpu.sync_copy(
            acc.at[pl.ds(s_ax * rows_per_tile, rows_per_tile)],
            deg_hbm.at[c_ax, pl.ds(s_ax * rows_per_tile, rows_per_tile)],
        )

    return deg_kernel


def _make_agg_kernel(e_pad, n_pad, d):
    cpt = e_pad // (NW * K)
    rows_per_tile = n_pad // N_SUBCORES
    groups = d // LANES

    @functools.partial(
        pl.kernel,
        out_type=jax.ShapeDtypeStruct((N_CORES, n_pad, d), jnp.float32),
        mesh=_sc_mesh(),
        compiler_params=pltpu.CompilerParams(needs_layout_passes=False),
        scratch_types=(
            [pltpu.VMEM((K,), jnp.int32) for _ in range(NIDX)]
            + [pltpu.VMEM((K,), jnp.int32) for _ in range(NIDX)]
            + [pltpu.VMEM((K,), jnp.float32) for _ in range(NIDX)]
            + [pltpu.VMEM((K, d), jnp.float32) for _ in range(NBUF)]
            + [pltpu.SemaphoreType.DMA for _ in range(NIDX + 2 * NBUF)]
            + [pltpu.VMEM_SHARED((n_pad, d), jnp.float32)]
        ),
    )
    def agg_kernel(row_hbm, col_hbm, ew_hbm, y_hbm, agg_hbm, *refs):
        rowb = refs[0:NIDX]
        colb = refs[NIDX:2 * NIDX]
        ewb = refs[2 * NIDX:3 * NIDX]
        rows = refs[3 * NIDX:3 * NIDX + NBUF]
        isem = refs[3 * NIDX + NBUF:3 * NIDX + NBUF + NIDX]
        gsem = refs[3 * NIDX + NBUF + NIDX:3 * NIDX + NBUF + NIDX + NBUF]
        ssem = refs[3 * NIDX + NBUF + NIDX + NBUF:
                    3 * NIDX + NBUF + NIDX + 2 * NBUF]
        acc = refs[3 * NIDX + NBUF + NIDX + 2 * NBUF]

        c_ax = lax.axis_index("c")
        s_ax = lax.axis_index("s")
        wid = c_ax * N_SUBCORES + s_ax
        tile_base = wid * cpt * K

        @pl.loop(0, K)
        def _zero(i):
            for f in range(groups):
                rows[0][i, pl.ds(f * LANES, LANES)] = jnp.zeros(
                    (LANES,), jnp.float32)

        @pl.loop(0, rows_per_tile // K)
        def _zacc(t):
            pltpu.sync_copy(rows[0], acc.at[pl.ds(s_ax * rows_per_tile + t * K, K)])

        plsc.subcore_barrier()

        def issue_idx(ch, b):
            base = tile_base + ch * K
            pltpu.async_copy(row_hbm.at[pl.ds(base, K)], rowb[b], isem[b])
            pltpu.async_copy(col_hbm.at[pl.ds(base, K)], colb[b], isem[b])
            pltpu.async_copy(ew_hbm.at[pl.ds(base, K)], ewb[b], isem[b])

        def wait_idx(b):
            pltpu.make_async_copy(row_hbm.at[pl.ds(0, K)], rowb[b], isem[b]).wait()
            pltpu.make_async_copy(col_hbm.at[pl.ds(0, K)], colb[b], isem[b]).wait()
            pltpu.make_async_copy(ew_hbm.at[pl.ds(0, K)], ewb[b], isem[b]).wait()

        def issue_gather(b8, b4):
            pltpu.async_copy(y_hbm.at[rowb[b8]], rows[b4], gsem[b4])

        def wait_gather(b4):
            pltpu.make_async_copy(
                y_hbm.at[rowb[0]], rows[b4], gsem[b4]).wait()

        def wait_scat(b8, b4):
            pltpu.make_async_copy(rows[b4], acc.at[colb[b8]], ssem[b4]).wait()

        for ch in range(4):
            issue_idx(ch, ch)
        for ch in range(2):
            wait_idx(ch)
            issue_gather(ch, ch)

        @pl.loop(0, cpt, step=NIDX)
        def _main(g):
            for b in range(NIDX):
                ch = g + b
                b4 = b % NBUF
                tg8, tg4 = (b + 2) % NIDX, (b + 2) % NBUF
                ti = (b + 4) % NIDX

                wait_gather(b4)

                @pl.when(ch + 2 < cpt)
                def _prep_gather():
                    @pl.when(ch >= 2)
                    def _drain():
                        wait_scat(tg8, tg4)
                    wait_idx(tg8)
                    issue_gather(tg8, tg4)

                @pl.when(ch + 4 < cpt)
                def _prep_idx():
                    issue_idx(ch + 4, ti)

                @pl.loop(0, K)
                def _scale(j):
                    jv = jnp.broadcast_to(j, (LANES,)).astype(jnp.int32)
                    sv = plsc.load_gather(ewb[b], [jv])
                    for f in range(groups):
                        rows[b4][j, pl.ds(f * LANES, LANES)] = (
                            rows[b4][j, pl.ds(f * LANES, LANES)] * sv
                        )

                pltpu.async_copy(rows[b4], acc.at[colb[b]], ssem[b4], add=True)

        for b in range(NBUF):
            ch = cpt - NBUF + b
            wait_scat(ch % NIDX, ch % NBUF)

        plsc.subcore_barrier()
        pltpu.sync_copy(
            acc.at[pl.ds(s_ax * rows_per_tile, rows_per_tile)],
            agg_hbm.at[c_ax, pl.ds(s_ax * rows_per_tile, rows_per_tile)],
        )

    return agg_kernel


def _tc_xw_dis(x_p, W, deg3, n_pad, d):
    nb = n_pad // BR

    def body(xb, wb, degb, yb, disb):
        dcol = degb[0] + degb[1] + 1.0
        dis = jnp.where(dcol > 0, lax.rsqrt(dcol), 0.0)
        xw = jnp.dot(xb[...], wb[...], preferred_element_type=jnp.float32)
        yb[...] = xw * dis
        disb[...] = dis

    return pl.pallas_call(
        body,
        grid=(nb,),
        in_specs=[
            pl.BlockSpec((BR, d), lambda i: (i, 0)),
            pl.BlockSpec((d, d), lambda i: (0, 0)),
            pl.BlockSpec((N_CORES, BR, 1), lambda i: (0, i, 0)),
        ],
        out_specs=[
            pl.BlockSpec((BR, d), lambda i: (i, 0)),
            pl.BlockSpec((BR, 1), lambda i: (i, 0)),
        ],
        out_shape=[
            jax.ShapeDtypeStruct((n_pad, d), jnp.float32),
            jax.ShapeDtypeStruct((n_pad, 1), jnp.float32),
        ],
    )(x_p, W, deg3)


def _tc_final(agg, y, dis, b2, n_pad, d):
    nb = n_pad // BR

    def body(aggb, yb, disb, bb, ob):
        s = (aggb[0] + aggb[1] + yb[...]) * disb[...] + bb[...]
        ob[...] = jnp.maximum(s, 0.0)

    return pl.pallas_call(
        body,
        grid=(nb,),
        in_specs=[
            pl.BlockSpec((N_CORES, BR, d), lambda i: (0, i, 0)),
            pl.BlockSpec((BR, d), lambda i: (i, 0)),
            pl.BlockSpec((BR, 1), lambda i: (i, 0)),
            pl.BlockSpec((1, d), lambda i: (0, 0)),
        ],
        out_specs=pl.BlockSpec((BR, d), lambda i: (i, 0)),
        out_shape=jax.ShapeDtypeStruct((n_pad, d), jnp.float32),
    )(agg, y, dis, b2)


def kernel(x, edge_index, edge_weight, W, b):
    n, d = x.shape
    e = edge_index.shape[1]

    n_pad = ((n + NW * LANES - 1) // (NW * LANES)) * (NW * LANES)
    step = NW * K * NIDX
    e_pad = ((e + step - 1) // step) * step

    row = edge_index[0].astype(jnp.int32)
    col = edge_index[1].astype(jnp.int32)
    ew = edge_weight.astype(jnp.float32)
    if e_pad != e:
        pad = e_pad - e
        row = jnp.concatenate([row, jnp.zeros((pad,), jnp.int32)])
        col = jnp.concatenate([col, jnp.zeros((pad,), jnp.int32)])
        ew = jnp.concatenate([ew, jnp.zeros((pad,), jnp.float32)])
    x_p = x
    if n_pad != n:
        x_p = jnp.concatenate(
            [x, jnp.zeros((n_pad - n, d), jnp.float32)], axis=0)

    deg_p = _make_deg_kernel(e_pad, n_pad)(col, ew)
    deg3 = deg_p.reshape(N_CORES, n_pad, 1)
    y, dis = _tc_xw_dis(x_p, W, deg3, n_pad, d)
    agg = _make_agg_kernel(e_pad, n_pad, d)(row, col, ew, y)
    out = _tc_final(agg, y, dis, b.reshape(1, d), n_pad, d)
    return out[:n]


# speedup vs baseline: 13.6733x; 1.0681x over previous
import functools

import jax
import jax.numpy as jnp
from jax import lax
from jax.experimental import pallas as pl
from jax.experimental.pallas import tpu as pltpu
from jax.experimental.pallas import tpu_sc as plsc

N_CORES = 2
N_SUBCORES = 16
NW = N_CORES * N_SUBCORES
LANES = 16
K = 64
NBUF = 4
NIDX = 8
BR = 256


def _sc_mesh():
    return plsc.VectorSubcoreMesh(core_axis_name="c", subcore_axis_name="s")


def _make_deg_kernel(e_pad, n_pad):
    cpt = e_pad // (NW * K)
    rows_per_tile = n_pad // N_SUBCORES

    @functools.partial(
        pl.kernel,
        out_type=jax.ShapeDtypeStruct((N_CORES, n_pad), jnp.float32),
        mesh=_sc_mesh(),
        compiler_params=pltpu.CompilerParams(needs_layout_passes=False),
        scratch_types=(
            [pltpu.VMEM((K,), jnp.int32) for _ in range(NBUF)]
            + [pltpu.VMEM((K,), jnp.float32) for _ in range(NBUF)]
            + [pltpu.VMEM((rows_per_tile,), jnp.float32)]
            + [pltpu.SemaphoreType.DMA for _ in range(2 * NBUF)]
            + [pltpu.VMEM_SHARED((n_pad,), jnp.float32)]
        ),
    )
    def deg_kernel(col_hbm, ew_hbm, deg_hbm, *refs):
        colb = refs[0:NBUF]
        ewb = refs[NBUF:2 * NBUF]
        zbuf = refs[2 * NBUF]
        isem = refs[2 * NBUF + 1:2 * NBUF + 1 + NBUF]
        ssem = refs[2 * NBUF + 1 + NBUF:2 * NBUF + 1 + 2 * NBUF]
        acc = refs[2 * NBUF + 1 + 2 * NBUF]

        c_ax = lax.axis_index("c")
        s_ax = lax.axis_index("s")
        wid = c_ax * N_SUBCORES + s_ax
        tile_base = wid * cpt * K

        @pl.loop(0, rows_per_tile // LANES)
        def _zero(i):
            zbuf[pl.ds(i * LANES, LANES)] = jnp.zeros((LANES,), jnp.float32)

        pltpu.sync_copy(zbuf, acc.at[pl.ds(s_ax * rows_per_tile, rows_per_tile)])
        plsc.subcore_barrier()

        def issue_idx(ch, b):
            base = tile_base + ch * K
            pltpu.async_copy(col_hbm.at[pl.ds(base, K)], colb[b], isem[b])
            pltpu.async_copy(ew_hbm.at[pl.ds(base, K)], ewb[b], isem[b])

        def wait_idx(b):
            pltpu.make_async_copy(col_hbm.at[pl.ds(0, K)], colb[b], isem[b]).wait()
            pltpu.make_async_copy(ew_hbm.at[pl.ds(0, K)], ewb[b], isem[b]).wait()

        def wait_scat(b):
            pltpu.make_async_copy(ewb[b], acc.at[colb[b]], ssem[b]).wait()

        issue_idx(0, 0)
        issue_idx(1, 1)

        @pl.loop(0, cpt, step=NBUF)
        def _main(g):
            for b in range(NBUF):
                ch = g + b
                t = (b + 2) % NBUF

                @pl.when(ch + 2 < cpt)
                def _prep():
                    @pl.when(ch >= 2)
                    def _drain():
                        wait_scat(t)
                    issue_idx(ch + 2, t)

                wait_idx(b)
                pltpu.async_copy(ewb[b], acc.at[colb[b]], ssem[b], add=True)

        for b in range(NBUF):
            wait_scat(b)

        plsc.subcore_barrier()
        pltpu.sync_copy(
            acc.at[pl.ds(s_ax * rows_per_tile, rows_per_tile)],
            deg_hbm.at[c_ax, pl.ds(s_ax * rows_per_tile, rows_per_tile)],
        )

    return deg_kernel


def _make_agg_kernel(e_pad, n_pad, d):
    total_pt = e_pad // (N_SUBCORES * K)
    cpt0 = (total_pt * 25 // 32) // NIDX * NIDX
    cpt1 = total_pt - cpt0
    assert cpt1 % NIDX == 0 and cpt1 > 0
    rows_per_tile = n_pad // N_SUBCORES
    groups = d // LANES

    @functools.partial(
        pl.kernel,
        out_type=jax.ShapeDtypeStruct((N_CORES, n_pad, d), jnp.float32),
        mesh=_sc_mesh(),
        compiler_params=pltpu.CompilerParams(needs_layout_passes=False),
        scratch_types=(
            [pltpu.VMEM((K,), jnp.int32) for _ in range(NIDX)]
            + [pltpu.VMEM((K,), jnp.int32) for _ in range(NIDX)]
            + [pltpu.VMEM((K,), jnp.float32) for _ in range(NIDX)]
            + [pltpu.VMEM((K, d), jnp.float32) for _ in range(NBUF)]
            + [pltpu.SemaphoreType.DMA for _ in range(NIDX + 2 * NBUF)]
            + [pltpu.VMEM_SHARED((n_pad, d), jnp.float32)]
        ),
    )
    def agg_kernel(row_hbm, col_hbm, ew_hbm, y_hbm, agg_hbm, *refs):
        rowb = refs[0:NIDX]
        colb = refs[NIDX:2 * NIDX]
        ewb = refs[2 * NIDX:3 * NIDX]
        rows = refs[3 * NIDX:3 * NIDX + NBUF]
        isem = refs[3 * NIDX + NBUF:3 * NIDX + NBUF + NIDX]
        gsem = refs[3 * NIDX + NBUF + NIDX:3 * NIDX + NBUF + NIDX + NBUF]
        ssem = refs[3 * NIDX + NBUF + NIDX + NBUF:
                    3 * NIDX + NBUF + NIDX + 2 * NBUF]
        acc = refs[3 * NIDX + NBUF + NIDX + 2 * NBUF]

        c_ax = lax.axis_index("c")
        s_ax = lax.axis_index("s")
        cpt = jnp.where(c_ax == 0, cpt0, cpt1)
        tile_base = jnp.where(
            c_ax == 0,
            s_ax * (cpt0 * K),
            N_SUBCORES * cpt0 * K + s_ax * (cpt1 * K),
        )

        @pl.loop(0, K)
        def _zero(i):
            for f in range(groups):
                rows[0][i, pl.ds(f * LANES, LANES)] = jnp.zeros(
                    (LANES,), jnp.float32)

        @pl.loop(0, rows_per_tile // K)
        def _zacc(t):
            pltpu.sync_copy(rows[0], acc.at[pl.ds(s_ax * rows_per_tile + t * K, K)])

        plsc.subcore_barrier()

        def issue_idx(ch, b):
            base = tile_base + ch * K
            pltpu.async_copy(row_hbm.at[pl.ds(base, K)], rowb[b], isem[b])
            pltpu.async_copy(col_hbm.at[pl.ds(base, K)], colb[b], isem[b])
            pltpu.async_copy(ew_hbm.at[pl.ds(base, K)], ewb[b], isem[b])

        def wait_idx(b):
            pltpu.make_async_copy(row_hbm.at[pl.ds(0, K)], rowb[b], isem[b]).wait()
            pltpu.make_async_copy(col_hbm.at[pl.ds(0, K)], colb[b], isem[b]).wait()
            pltpu.make_async_copy(ew_hbm.at[pl.ds(0, K)], ewb[b], isem[b]).wait()

        def issue_gather(b8, b4):
            pltpu.async_copy(y_hbm.at[rowb[b8]], rows[b4], gsem[b4])

        def wait_gather(b4):
            pltpu.make_async_copy(
                y_hbm.at[rowb[0]], rows[b4], gsem[b4]).wait()

        def wait_scat(b8, b4):
            pltpu.make_async_copy(rows[b4], acc.at[colb[b8]], ssem[b4]).wait()

        for ch in range(4):
            issue_idx(ch, ch)
        for ch in range(2):
            wait_idx(ch)
            issue_gather(ch, ch)

        @pl.loop(0, cpt, step=NIDX)
        def _main(g):
            for b in range(NIDX):
                ch = g + b
                b4 = b % NBUF
                tg8, tg4 = (b + 2) % NIDX, (b + 2) % NBUF
                ti = (b + 4) % NIDX

                wait_gather(b4)

                @pl.when(ch + 2 < cpt)
                def _prep_gather():
                    @pl.when(ch >= 2)
                    def _drain():
                        wait_scat(tg8, tg4)
                    wait_idx(tg8)
                    issue_gather(tg8, tg4)

                @pl.when(ch + 4 < cpt)
                def _prep_idx():
                    issue_idx(ch + 4, ti)

                @pl.loop(0, K)
                def _scale(j):
                    jv = jnp.broadcast_to(j, (LANES,)).astype(jnp.int32)
                    sv = plsc.load_gather(ewb[b], [jv])
                    for f in range(groups):
                        rows[b4][j, pl.ds(f * LANES, LANES)] = (
                            rows[b4][j, pl.ds(f * LANES, LANES)] * sv
                        )

                pltpu.async_copy(rows[b4], acc.at[colb[b]], ssem[b4], add=True)

        for b in range(NBUF):
            wait_scat(NBUF + b, b)

        plsc.subcore_barrier()
        pltpu.sync_copy(
            acc.at[pl.ds(s_ax * rows_per_tile, rows_per_tile)],
            agg_hbm.at[c_ax, pl.ds(s_ax * rows_per_tile, rows_per_tile)],
        )

    return agg_kernel


def _tc_xw_dis(x_p, W, deg3, n_pad, d):
    nb = n_pad // BR

    def body(xb, wb, degb, yb, disb):
        dcol = degb[0] + degb[1] + 1.0
        dis = jnp.where(dcol > 0, lax.rsqrt(dcol), 0.0)
        xw = jnp.dot(xb[...], wb[...], preferred_element_type=jnp.float32)
        yb[...] = xw * dis
        disb[...] = dis

    return pl.pallas_call(
        body,
        grid=(nb,),
        in_specs=[
            pl.BlockSpec((BR, d), lambda i: (i, 0)),
            pl.BlockSpec((d, d), lambda i: (0, 0)),
            pl.BlockSpec((N_CORES, BR, 1), lambda i: (0, i, 0)),
        ],
        out_specs=[
            pl.BlockSpec((BR, d), lambda i: (i, 0)),
            pl.BlockSpec((BR, 1), lambda i: (i, 0)),
        ],
        out_shape=[
            jax.ShapeDtypeStruct((n_pad, d), jnp.float32),
            jax.ShapeDtypeStruct((n_pad, 1), jnp.float32),
        ],
    )(x_p, W, deg3)


def _tc_final(agg, y, dis, b2, n_pad, d):
    nb = n_pad // BR

    def body(aggb, yb, disb, bb, ob):
        s = (aggb[0] + aggb[1] + yb[...]) * disb[...] + bb[...]
        ob[...] = jnp.maximum(s, 0.0)

    return pl.pallas_call(
        body,
        grid=(nb,),
        in_specs=[
            pl.BlockSpec((N_CORES, BR, d), lambda i: (0, i, 0)),
            pl.BlockSpec((BR, d), lambda i: (i, 0)),
            pl.BlockSpec((BR, 1), lambda i: (i, 0)),
            pl.BlockSpec((1, d), lambda i: (0, 0)),
        ],
        out_specs=pl.BlockSpec((BR, d), lambda i: (i, 0)),
        out_shape=jax.ShapeDtypeStruct((n_pad, d), jnp.float32),
    )(agg, y, dis, b2)


def kernel(x, edge_index, edge_weight, W, b):
    n, d = x.shape
    e = edge_index.shape[1]

    n_pad = ((n + NW * LANES - 1) // (NW * LANES)) * (NW * LANES)
    step = NW * K * NIDX
    e_pad = ((e + step - 1) // step) * step

    row = edge_index[0].astype(jnp.int32)
    col = edge_index[1].astype(jnp.int32)
    ew = edge_weight.astype(jnp.float32)
    if e_pad != e:
        pad = e_pad - e
        row = jnp.concatenate([row, jnp.zeros((pad,), jnp.int32)])
        col = jnp.concatenate([col, jnp.zeros((pad,), jnp.int32)])
        ew = jnp.concatenate([ew, jnp.zeros((pad,), jnp.float32)])
    x_p = x
    if n_pad != n:
        x_p = jnp.concatenate(
            [x, jnp.zeros((n_pad - n, d), jnp.float32)], axis=0)

    deg_p = _make_deg_kernel(e_pad, n_pad)(col, ew)
    deg3 = deg_p.reshape(N_CORES, n_pad, 1)
    y, dis = _tc_xw_dis(x_p, W, deg3, n_pad, d)
    agg = _make_agg_kernel(e_pad, n_pad, d)(row, col, ew, y)
    out = _tc_final(agg, y, dis, b.reshape(1, d), n_pad, d)
    return out[:n]


# speedup vs baseline: 31.6269x; 2.3130x over previous
import functools

import jax
import jax.numpy as jnp
from jax import lax
from jax.experimental import pallas as pl
from jax.experimental.pallas import tpu as pltpu
from jax.experimental.pallas import tpu_sc as plsc

N_CORES = 2
N_SUBCORES = 16
NW = N_CORES * N_SUBCORES
LANES = 16
K = 64
NBUF = 4
NIDX = 8
BR = 256


def _sc_mesh():
    return plsc.VectorSubcoreMesh(core_axis_name="c", subcore_axis_name="s")


def _make_deg_kernel(e_pad, n_pad):
    cpt = e_pad // (NW * K)
    rows_per_tile = n_pad // N_SUBCORES

    @functools.partial(
        pl.kernel,
        out_type=jax.ShapeDtypeStruct((N_CORES, n_pad), jnp.float32),
        mesh=_sc_mesh(),
        compiler_params=pltpu.CompilerParams(needs_layout_passes=False),
        scratch_types=(
            [pltpu.VMEM((K,), jnp.int32) for _ in range(NBUF)]
            + [pltpu.VMEM((K,), jnp.float32) for _ in range(NBUF)]
            + [pltpu.VMEM((rows_per_tile,), jnp.float32)]
            + [pltpu.SemaphoreType.DMA for _ in range(2 * NBUF)]
            + [pltpu.VMEM_SHARED((n_pad,), jnp.float32)]
        ),
    )
    def deg_kernel(col_hbm, ew_hbm, deg_hbm, *refs):
        colb = refs[0:NBUF]
        ewb = refs[NBUF:2 * NBUF]
        zbuf = refs[2 * NBUF]
        isem = refs[2 * NBUF + 1:2 * NBUF + 1 + NBUF]
        ssem = refs[2 * NBUF + 1 + NBUF:2 * NBUF + 1 + 2 * NBUF]
        acc = refs[2 * NBUF + 1 + 2 * NBUF]

        c_ax = lax.axis_index("c")
        s_ax = lax.axis_index("s")
        wid = c_ax * N_SUBCORES + s_ax
        tile_base = wid * cpt * K

        @pl.loop(0, rows_per_tile // LANES)
        def _zero(i):
            zbuf[pl.ds(i * LANES, LANES)] = jnp.zeros((LANES,), jnp.float32)

        pltpu.sync_copy(zbuf, acc.at[pl.ds(s_ax * rows_per_tile, rows_per_tile)])
        plsc.subcore_barrier()

        def issue_idx(ch, b):
            base = tile_base + ch * K
            pltpu.async_copy(col_hbm.at[pl.ds(base, K)], colb[b], isem[b])
            pltpu.async_copy(ew_hbm.at[pl.ds(base, K)], ewb[b], isem[b])

        def wait_idx(b):
            pltpu.make_async_copy(col_hbm.at[pl.ds(0, K)], colb[b], isem[b]).wait()
            pltpu.make_async_copy(ew_hbm.at[pl.ds(0, K)], ewb[b], isem[b]).wait()

        def wait_scat(b):
            pltpu.make_async_copy(ewb[b], acc.at[colb[b]], ssem[b]).wait()

        issue_idx(0, 0)
        issue_idx(1, 1)

        @pl.loop(0, cpt, step=NBUF)
        def _main(g):
            for b in range(NBUF):
                ch = g + b
                t = (b + 2) % NBUF

                @pl.when(ch + 2 < cpt)
                def _prep():
                    @pl.when(ch >= 2)
                    def _drain():
                        wait_scat(t)
                    issue_idx(ch + 2, t)

                wait_idx(b)
                pltpu.async_copy(ewb[b], acc.at[colb[b]], ssem[b], add=True)

        for b in range(NBUF):
            wait_scat(b)

        plsc.subcore_barrier()
        pltpu.sync_copy(
            acc.at[pl.ds(s_ax * rows_per_tile, rows_per_tile)],
            deg_hbm.at[c_ax, pl.ds(s_ax * rows_per_tile, rows_per_tile)],
        )

    return deg_kernel


def _make_agg_kernel(e_pad, n_pad, d):
    total_pt = e_pad // (N_SUBCORES * K)
    cpt0 = (total_pt // 2) // NIDX * NIDX
    cpt1 = total_pt - cpt0
    assert cpt1 % NIDX == 0 and cpt1 > 0
    rows_per_tile = n_pad // N_SUBCORES
    groups = d // LANES

    @functools.partial(
        pl.kernel,
        out_type=jax.ShapeDtypeStruct((N_CORES, n_pad, d), jnp.float32),
        mesh=_sc_mesh(),
        compiler_params=pltpu.CompilerParams(needs_layout_passes=False),
        scratch_types=(
            [pltpu.VMEM((K,), jnp.int32) for _ in range(NIDX)]
            + [pltpu.VMEM((K,), jnp.int32) for _ in range(NIDX)]
            + [pltpu.VMEM((K,), jnp.float32) for _ in range(NIDX)]
            + [pltpu.VMEM((K, d), jnp.float32) for _ in range(NBUF)]
            + [pltpu.SemaphoreType.DMA for _ in range(NIDX + 2 * NBUF)]
            + [pltpu.VMEM_SHARED((n_pad, d), jnp.float32)]
        ),
    )
    def agg_kernel(row_hbm, col_hbm, ew_hbm, y_hbm, agg_hbm, *refs):
        rowb = refs[0:NIDX]
        colb = refs[NIDX:2 * NIDX]
        ewb = refs[2 * NIDX:3 * NIDX]
        rows = refs[3 * NIDX:3 * NIDX + NBUF]
        isem = refs[3 * NIDX + NBUF:3 * NIDX + NBUF + NIDX]
        gsem = refs[3 * NIDX + NBUF + NIDX:3 * NIDX + NBUF + NIDX + NBUF]
        ssem = refs[3 * NIDX + NBUF + NIDX + NBUF:
                    3 * NIDX + NBUF + NIDX + 2 * NBUF]
        acc = refs[3 * NIDX + NBUF + NIDX + 2 * NBUF]

        c_ax = lax.axis_index("c")
        s_ax = lax.axis_index("s")
        cpt = jnp.where(c_ax == 0, cpt0, cpt1)
        tile_base = jnp.where(
            c_ax == 0,
            s_ax * (cpt0 * K),
            N_SUBCORES * cpt0 * K + s_ax * (cpt1 * K),
        )

        @pl.loop(0, K)
        def _zero(i):
            for f in range(groups):
                rows[0][i, pl.ds(f * LANES, LANES)] = jnp.zeros(
                    (LANES,), jnp.float32)

        @pl.loop(0, rows_per_tile // K)
        def _zacc(t):
            pltpu.sync_copy(rows[0], acc.at[pl.ds(s_ax * rows_per_tile + t * K, K)])

        plsc.subcore_barrier()

        def issue_idx(ch, b):
            base = tile_base + ch * K
            pltpu.async_copy(row_hbm.at[pl.ds(base, K)], rowb[b], isem[b])
            pltpu.async_copy(col_hbm.at[pl.ds(base, K)], colb[b], isem[b])
            pltpu.async_copy(ew_hbm.at[pl.ds(base, K)], ewb[b], isem[b])

        def wait_idx(b):
            pltpu.make_async_copy(row_hbm.at[pl.ds(0, K)], rowb[b], isem[b]).wait()
            pltpu.make_async_copy(col_hbm.at[pl.ds(0, K)], colb[b], isem[b]).wait()
            pltpu.make_async_copy(ew_hbm.at[pl.ds(0, K)], ewb[b], isem[b]).wait()

        def issue_gather(b8, b4):
            pltpu.async_copy(y_hbm.at[rowb[b8]], rows[b4], gsem[b4])

        def wait_gather(b4):
            pltpu.make_async_copy(
                y_hbm.at[rowb[0]], rows[b4], gsem[b4]).wait()

        def wait_scat(b8, b4):
            pltpu.make_async_copy(rows[b4], acc.at[colb[b8]], ssem[b4]).wait()

        for ch in range(4):
            issue_idx(ch, ch)
        for ch in range(2):
            wait_idx(ch)
            issue_gather(ch, ch)

        @pl.loop(0, cpt, step=NIDX)
        def _main(g):
            for b in range(NIDX):
                ch = g + b
                b4 = b % NBUF
                tg8, tg4 = (b + 2) % NIDX, (b + 2) % NBUF
                ti = (b + 4) % NIDX

                wait_gather(b4)

                @pl.when(ch + 2 < cpt)
                def _prep_gather():
                    @pl.when(ch >= 2)
                    def _drain():
                        wait_scat(tg8, tg4)
                    wait_idx(tg8)
                    issue_gather(tg8, tg4)

                @pl.when(ch + 4 < cpt)
                def _prep_idx():
                    issue_idx(ch + 4, ti)

                @pl.loop(0, K)
                def _scale(j):
                    jv = jnp.broadcast_to(j, (LANES,)).astype(jnp.int32)
                    sv = plsc.load_gather(ewb[b], [jv])
                    for f in range(groups):
                        rows[b4][j, pl.ds(f * LANES, LANES)] = (
                            rows[b4][j, pl.ds(f * LANES, LANES)] * sv
                        )

                pltpu.async_copy(rows[b4], acc.at[colb[b]], ssem[b4], add=True)

        for b in range(NBUF):
            wait_scat(NBUF + b, b)

        plsc.subcore_barrier()
        pltpu.sync_copy(
            acc.at[pl.ds(s_ax * rows_per_tile, rows_per_tile)],
            agg_hbm.at[c_ax, pl.ds(s_ax * rows_per_tile, rows_per_tile)],
        )

    return agg_kernel


def _tc_xw_dis(x_p, W, deg3, n_pad, d):
    nb = n_pad // BR

    def body(xb, wb, degb, yb, disb):
        dcol = degb[0] + degb[1] + 1.0
        dis = jnp.where(dcol > 0, lax.rsqrt(dcol), 0.0)
        xw = jnp.dot(xb[...], wb[...], preferred_element_type=jnp.float32)
        yb[...] = xw * dis
        disb[...] = dis

    return pl.pallas_call(
        body,
        grid=(nb,),
        in_specs=[
            pl.BlockSpec((BR, d), lambda i: (i, 0)),
            pl.BlockSpec((d, d), lambda i: (0, 0)),
            pl.BlockSpec((N_CORES, BR, 1), lambda i: (0, i, 0)),
        ],
        out_specs=[
            pl.BlockSpec((BR, d), lambda i: (i, 0)),
            pl.BlockSpec((BR, 1), lambda i: (i, 0)),
        ],
        out_shape=[
            jax.ShapeDtypeStruct((n_pad, d), jnp.float32),
            jax.ShapeDtypeStruct((n_pad, 1), jnp.float32),
        ],
    )(x_p, W, deg3)


def _tc_final(agg, y, dis, b2, n_pad, d):
    nb = n_pad // BR

    def body(aggb, yb, disb, bb, ob):
        s = (aggb[0] + aggb[1] + yb[...]) * disb[...] + bb[...]
        ob[...] = jnp.maximum(s, 0.0)

    return pl.pallas_call(
        body,
        grid=(nb,),
        in_specs=[
            pl.BlockSpec((N_CORES, BR, d), lambda i: (0, i, 0)),
            pl.BlockSpec((BR, d), lambda i: (i, 0)),
            pl.BlockSpec((BR, 1), lambda i: (i, 0)),
            pl.BlockSpec((1, d), lambda i: (0, 0)),
        ],
        out_specs=pl.BlockSpec((BR, d), lambda i: (i, 0)),
        out_shape=jax.ShapeDtypeStruct((n_pad, d), jnp.float32),
    )(agg, y, dis, b2)


def kernel(x, edge_index, edge_weight, W, b):
    n, d = x.shape
    e = edge_index.shape[1]

    n_pad = ((n + NW * LANES - 1) // (NW * LANES)) * (NW * LANES)
    step = NW * K * NIDX
    e_pad = ((e + step - 1) // step) * step

    row = edge_index[0].astype(jnp.int32)
    col = edge_index[1].astype(jnp.int32)
    ew = edge_weight.astype(jnp.float32)
    if e_pad != e:
        pad = e_pad - e
        spread = (jnp.arange(pad, dtype=jnp.int32) * LANES) % n
        row = jnp.concatenate([row, spread])
        col = jnp.concatenate([col, spread])
        ew = jnp.concatenate([ew, jnp.zeros((pad,), jnp.float32)])
    x_p = x
    if n_pad != n:
        x_p = jnp.concatenate(
            [x, jnp.zeros((n_pad - n, d), jnp.float32)], axis=0)

    deg_p = _make_deg_kernel(e_pad, n_pad)(col, ew)
    deg3 = deg_p.reshape(N_CORES, n_pad, 1)
    y, dis = _tc_xw_dis(x_p, W, deg3, n_pad, d)
    agg = _make_agg_kernel(e_pad, n_pad, d)(row, col, ew, y)
    out = _tc_final(agg, y, dis, b.reshape(1, d), n_pad, d)
    return out[:n]


# speedup vs baseline: 37.5683x; 1.1879x over previous
import functools

import jax
import jax.numpy as jnp
from jax import lax
from jax.experimental import pallas as pl
from jax.experimental.pallas import tpu as pltpu
from jax.experimental.pallas import tpu_sc as plsc

N_CORES = 2
N_SUBCORES = 16
NW = N_CORES * N_SUBCORES
LANES = 16
K = 64
NBUF = 4
NIDX = 8
BR = 1024


def _sc_mesh():
    return plsc.VectorSubcoreMesh(core_axis_name="c", subcore_axis_name="s")


def _make_deg_kernel(e_pad, n_pad):
    cpt = e_pad // (NW * K)
    rows_per_tile = n_pad // N_SUBCORES

    @functools.partial(
        pl.kernel,
        out_type=jax.ShapeDtypeStruct((N_CORES, n_pad), jnp.float32),
        mesh=_sc_mesh(),
        compiler_params=pltpu.CompilerParams(needs_layout_passes=False),
        scratch_types=(
            [pltpu.VMEM((K,), jnp.int32) for _ in range(NBUF)]
            + [pltpu.VMEM((K,), jnp.float32) for _ in range(NBUF)]
            + [pltpu.VMEM((rows_per_tile,), jnp.float32)]
            + [pltpu.SemaphoreType.DMA for _ in range(2 * NBUF)]
            + [pltpu.VMEM_SHARED((n_pad,), jnp.float32)]
        ),
    )
    def deg_kernel(col_hbm, ew_hbm, deg_hbm, *refs):
        colb = refs[0:NBUF]
        ewb = refs[NBUF:2 * NBUF]
        zbuf = refs[2 * NBUF]
        isem = refs[2 * NBUF + 1:2 * NBUF + 1 + NBUF]
        ssem = refs[2 * NBUF + 1 + NBUF:2 * NBUF + 1 + 2 * NBUF]
        acc = refs[2 * NBUF + 1 + 2 * NBUF]

        c_ax = lax.axis_index("c")
        s_ax = lax.axis_index("s")
        wid = c_ax * N_SUBCORES + s_ax
        tile_base = wid * cpt * K

        @pl.loop(0, rows_per_tile // LANES)
        def _zero(i):
            zbuf[pl.ds(i * LANES, LANES)] = jnp.zeros((LANES,), jnp.float32)

        pltpu.sync_copy(zbuf, acc.at[pl.ds(s_ax * rows_per_tile, rows_per_tile)])
        plsc.subcore_barrier()

        def issue_idx(ch, b):
            base = tile_base + ch * K
            pltpu.async_copy(col_hbm.at[pl.ds(base, K)], colb[b], isem[b])
            pltpu.async_copy(ew_hbm.at[pl.ds(base, K)], ewb[b], isem[b])

        def wait_idx(b):
            pltpu.make_async_copy(col_hbm.at[pl.ds(0, K)], colb[b], isem[b]).wait()
            pltpu.make_async_copy(ew_hbm.at[pl.ds(0, K)], ewb[b], isem[b]).wait()

        def wait_scat(b):
            pltpu.make_async_copy(ewb[b], acc.at[colb[b]], ssem[b]).wait()

        issue_idx(0, 0)
        issue_idx(1, 1)

        @pl.loop(0, cpt, step=NBUF)
        def _main(g):
            for b in range(NBUF):
                ch = g + b
                t = (b + 2) % NBUF

                @pl.when(ch + 2 < cpt)
                def _prep():
                    @pl.when(ch >= 2)
                    def _drain():
                        wait_scat(t)
                    issue_idx(ch + 2, t)

                wait_idx(b)
                pltpu.async_copy(ewb[b], acc.at[colb[b]], ssem[b], add=True)

        for b in range(NBUF):
            wait_scat(b)

        plsc.subcore_barrier()
        pltpu.sync_copy(
            acc.at[pl.ds(s_ax * rows_per_tile, rows_per_tile)],
            deg_hbm.at[c_ax, pl.ds(s_ax * rows_per_tile, rows_per_tile)],
        )

    return deg_kernel


def _make_agg_kernel(e_pad, n_pad, d):
    total_pt = e_pad // (N_SUBCORES * K)
    cpt0 = (total_pt // 2) // NIDX * NIDX
    cpt1 = total_pt - cpt0
    assert cpt1 % NIDX == 0 and cpt1 > 0
    rows_per_tile = n_pad // N_SUBCORES
    groups = d // LANES

    @functools.partial(
        pl.kernel,
        out_type=jax.ShapeDtypeStruct((N_CORES, n_pad, d), jnp.float32),
        mesh=_sc_mesh(),
        compiler_params=pltpu.CompilerParams(needs_layout_passes=False),
        scratch_types=(
            [pltpu.VMEM((K,), jnp.int32) for _ in range(NIDX)]
            + [pltpu.VMEM((K,), jnp.int32) for _ in range(NIDX)]
            + [pltpu.VMEM((K,), jnp.float32) for _ in range(NIDX)]
            + [pltpu.VMEM((K, d), jnp.float32) for _ in range(NBUF)]
            + [pltpu.SemaphoreType.DMA for _ in range(NIDX + 2 * NBUF)]
            + [pltpu.VMEM_SHARED((n_pad, d), jnp.float32)]
        ),
    )
    def agg_kernel(row_hbm, col_hbm, ew_hbm, y_hbm, agg_hbm, *refs):
        rowb = refs[0:NIDX]
        colb = refs[NIDX:2 * NIDX]
        ewb = refs[2 * NIDX:3 * NIDX]
        rows = refs[3 * NIDX:3 * NIDX + NBUF]
        isem = refs[3 * NIDX + NBUF:3 * NIDX + NBUF + NIDX]
        gsem = refs[3 * NIDX + NBUF + NIDX:3 * NIDX + NBUF + NIDX + NBUF]
        ssem = refs[3 * NIDX + NBUF + NIDX + NBUF:
                    3 * NIDX + NBUF + NIDX + 2 * NBUF]
        acc = refs[3 * NIDX + NBUF + NIDX + 2 * NBUF]

        c_ax = lax.axis_index("c")
        s_ax = lax.axis_index("s")
        cpt = jnp.where(c_ax == 0, cpt0, cpt1)
        tile_base = jnp.where(
            c_ax == 0,
            s_ax * (cpt0 * K),
            N_SUBCORES * cpt0 * K + s_ax * (cpt1 * K),
        )

        @pl.loop(0, K)
        def _zero(i):
            for f in range(groups):
                rows[0][i, pl.ds(f * LANES, LANES)] = jnp.zeros(
                    (LANES,), jnp.float32)

        @pl.loop(0, rows_per_tile // K)
        def _zacc(t):
            pltpu.sync_copy(rows[0], acc.at[pl.ds(s_ax * rows_per_tile + t * K, K)])

        plsc.subcore_barrier()

        def issue_idx(ch, b):
            base = tile_base + ch * K
            pltpu.async_copy(row_hbm.at[pl.ds(base, K)], rowb[b], isem[b])
            pltpu.async_copy(col_hbm.at[pl.ds(base, K)], colb[b], isem[b])
            pltpu.async_copy(ew_hbm.at[pl.ds(base, K)], ewb[b], isem[b])

        def wait_idx(b):
            pltpu.make_async_copy(row_hbm.at[pl.ds(0, K)], rowb[b], isem[b]).wait()
            pltpu.make_async_copy(col_hbm.at[pl.ds(0, K)], colb[b], isem[b]).wait()
            pltpu.make_async_copy(ew_hbm.at[pl.ds(0, K)], ewb[b], isem[b]).wait()

        def issue_gather(b8, b4):
            pltpu.async_copy(y_hbm.at[rowb[b8]], rows[b4], gsem[b4])

        def wait_gather(b4):
            pltpu.make_async_copy(
                y_hbm.at[rowb[0]], rows[b4], gsem[b4]).wait()

        def wait_scat(b8, b4):
            pltpu.make_async_copy(rows[b4], acc.at[colb[b8]], ssem[b4]).wait()

        for ch in range(4):
            issue_idx(ch, ch)
        for ch in range(2):
            wait_idx(ch)
            issue_gather(ch, ch)

        @pl.loop(0, cpt, step=NIDX)
        def _main(g):
            for b in range(NIDX):
                ch = g + b
                b4 = b % NBUF
                tg8, tg4 = (b + 2) % NIDX, (b + 2) % NBUF
                ti = (b + 4) % NIDX

                wait_gather(b4)

                @pl.when(ch + 2 < cpt)
                def _prep_gather():
                    @pl.when(ch >= 2)
                    def _drain():
                        wait_scat(tg8, tg4)
                    wait_idx(tg8)
                    issue_gather(tg8, tg4)

                @pl.when(ch + 4 < cpt)
                def _prep_idx():
                    issue_idx(ch + 4, ti)

                @pl.loop(0, K, unroll=2)
                def _scale(j):
                    jv = jnp.broadcast_to(j, (LANES,)).astype(jnp.int32)
                    sv = plsc.load_gather(ewb[b], [jv])
                    for f in range(groups):
                        rows[b4][j, pl.ds(f * LANES, LANES)] = (
                            rows[b4][j, pl.ds(f * LANES, LANES)] * sv
                        )

                pltpu.async_copy(rows[b4], acc.at[colb[b]], ssem[b4], add=True)

        for b in range(NBUF):
            wait_scat(NBUF + b, b)

        plsc.subcore_barrier()
        pltpu.sync_copy(
            acc.at[pl.ds(s_ax * rows_per_tile, rows_per_tile)],
            agg_hbm.at[c_ax, pl.ds(s_ax * rows_per_tile, rows_per_tile)],
        )

    return agg_kernel


def _tc_xw_dis(x_p, W, deg3, n_pad, d):
    nb = n_pad // BR
    n_in = x_p.shape[0]

    def body(xb, wb, degb, yb, disb):
        dcol = degb[0] + degb[1] + 1.0
        dis = jnp.where(dcol > 0, lax.rsqrt(dcol), 0.0)
        xw = jnp.dot(xb[...], wb[...], preferred_element_type=jnp.float32)
        yb[...] = xw * dis
        disb[...] = dis

    return pl.pallas_call(
        body,
        grid=(nb,),
        in_specs=[
            pl.BlockSpec((BR, d), lambda i: (i, 0)),
            pl.BlockSpec((d, d), lambda i: (0, 0)),
            pl.BlockSpec((N_CORES, BR, 1), lambda i: (0, i, 0)),
        ],
        out_specs=[
            pl.BlockSpec((BR, d), lambda i: (i, 0)),
            pl.BlockSpec((BR, 1), lambda i: (i, 0)),
        ],
        out_shape=[
            jax.ShapeDtypeStruct((n_pad, d), jnp.float32),
            jax.ShapeDtypeStruct((n_pad, 1), jnp.float32),
        ],
    )(x_p, W, deg3)


def _tc_final(agg, y, dis, b2, n, n_pad, d):
    nb = n_pad // BR

    def body(aggb, yb, disb, bb, ob):
        s = (aggb[0] + aggb[1] + yb[...]) * disb[...] + bb[...]
        ob[...] = jnp.maximum(s, 0.0)

    return pl.pallas_call(
        body,
        grid=(nb,),
        in_specs=[
            pl.BlockSpec((N_CORES, BR, d), lambda i: (0, i, 0)),
            pl.BlockSpec((BR, d), lambda i: (i, 0)),
            pl.BlockSpec((BR, 1), lambda i: (i, 0)),
            pl.BlockSpec((1, d), lambda i: (0, 0)),
        ],
        out_specs=pl.BlockSpec((BR, d), lambda i: (i, 0)),
        out_shape=jax.ShapeDtypeStruct((n, d), jnp.float32),
    )(agg, y, dis, b2)


def kernel(x, edge_index, edge_weight, W, b):
    n, d = x.shape
    e = edge_index.shape[1]

    n_pad = ((n + NW * LANES - 1) // (NW * LANES)) * (NW * LANES)
    step = NW * K * NIDX
    e_pad = ((e + step - 1) // step) * step

    row = edge_index[0].astype(jnp.int32)
    col = edge_index[1].astype(jnp.int32)
    ew = edge_weight.astype(jnp.float32)
    if e_pad != e:
        pad = e_pad - e
        spread = (jnp.arange(pad, dtype=jnp.int32) * LANES) % n
        row = jnp.concatenate([row, spread])
        col = jnp.concatenate([col, spread])
        ew = jnp.concatenate([ew, jnp.zeros((pad,), jnp.float32)])
    deg_p = _make_deg_kernel(e_pad, n_pad)(col, ew)
    deg3 = deg_p.reshape(N_CORES, n_pad, 1)
    y, dis = _tc_xw_dis(x, W, deg3, n_pad, d)
    agg = _make_agg_kernel(e_pad, n_pad, d)(row, col, ew, y)
    return _tc_final(agg, y, dis, b.reshape(1, d), n, n_pad, d)


# speedup vs baseline: 39.5957x; 1.0540x over previous
import functools

import jax
import jax.numpy as jnp
from jax import lax
from jax.experimental import pallas as pl
from jax.experimental.pallas import tpu as pltpu
from jax.experimental.pallas import tpu_sc as plsc

N_CORES = 2
N_SUBCORES = 16
NW = N_CORES * N_SUBCORES
LANES = 16
K = 64
KD = 128
NBUF = 4
NIDX = 8
BR = 1024


def _sc_mesh():
    return plsc.VectorSubcoreMesh(core_axis_name="c", subcore_axis_name="s")


def _make_deg_kernel(e_pad, n_pad):
    cpt = e_pad // (NW * KD)
    rows_per_tile = n_pad // N_SUBCORES

    @functools.partial(
        pl.kernel,
        out_type=jax.ShapeDtypeStruct((N_CORES, n_pad), jnp.float32),
        mesh=_sc_mesh(),
        compiler_params=pltpu.CompilerParams(needs_layout_passes=False),
        scratch_types=(
            [pltpu.VMEM((KD,), jnp.int32) for _ in range(NBUF)]
            + [pltpu.VMEM((KD,), jnp.float32) for _ in range(NBUF)]
            + [pltpu.VMEM((rows_per_tile,), jnp.float32)]
            + [pltpu.SemaphoreType.DMA for _ in range(2 * NBUF)]
            + [pltpu.VMEM_SHARED((n_pad,), jnp.float32)]
        ),
    )
    def deg_kernel(col_hbm, ew_hbm, deg_hbm, *refs):
        colb = refs[0:NBUF]
        ewb = refs[NBUF:2 * NBUF]
        zbuf = refs[2 * NBUF]
        isem = refs[2 * NBUF + 1:2 * NBUF + 1 + NBUF]
        ssem = refs[2 * NBUF + 1 + NBUF:2 * NBUF + 1 + 2 * NBUF]
        acc = refs[2 * NBUF + 1 + 2 * NBUF]

        c_ax = lax.axis_index("c")
        s_ax = lax.axis_index("s")
        wid = c_ax * N_SUBCORES + s_ax
        tile_base = wid * cpt * KD

        @pl.loop(0, rows_per_tile // LANES)
        def _zero(i):
            zbuf[pl.ds(i * LANES, LANES)] = jnp.zeros((LANES,), jnp.float32)

        pltpu.sync_copy(zbuf, acc.at[pl.ds(s_ax * rows_per_tile, rows_per_tile)])
        plsc.subcore_barrier()

        def issue_idx(ch, b):
            base = tile_base + ch * KD
            pltpu.async_copy(col_hbm.at[pl.ds(base, KD)], colb[b], isem[b])
            pltpu.async_copy(ew_hbm.at[pl.ds(base, KD)], ewb[b], isem[b])

        def wait_idx(b):
            pltpu.make_async_copy(col_hbm.at[pl.ds(0, KD)], colb[b], isem[b]).wait()
            pltpu.make_async_copy(ew_hbm.at[pl.ds(0, KD)], ewb[b], isem[b]).wait()

        def wait_scat(b):
            pltpu.make_async_copy(ewb[b], acc.at[colb[b]], ssem[b]).wait()

        issue_idx(0, 0)
        issue_idx(1, 1)

        @pl.loop(0, cpt, step=NBUF)
        def _main(g):
            for b in range(NBUF):
                ch = g + b
                t = (b + 2) % NBUF

                @pl.when(ch + 2 < cpt)
                def _prep():
                    @pl.when(ch >= 2)
                    def _drain():
                        wait_scat(t)
                    issue_idx(ch + 2, t)

                wait_idx(b)
                pltpu.async_copy(ewb[b], acc.at[colb[b]], ssem[b], add=True)

        for b in range(NBUF):
            wait_scat(b)

        plsc.subcore_barrier()
        pltpu.sync_copy(
            acc.at[pl.ds(s_ax * rows_per_tile, rows_per_tile)],
            deg_hbm.at[c_ax, pl.ds(s_ax * rows_per_tile, rows_per_tile)],
        )

    return deg_kernel


def _make_agg_kernel(e_pad, n_pad, d):
    total_pt = e_pad // (N_SUBCORES * K)
    cpt0 = (total_pt // 2) // NIDX * NIDX
    cpt1 = total_pt - cpt0
    assert cpt1 % NIDX == 0 and cpt1 > 0
    rows_per_tile = n_pad // N_SUBCORES
    groups = d // LANES

    @functools.partial(
        pl.kernel,
        out_type=jax.ShapeDtypeStruct((N_CORES, n_pad, d), jnp.float32),
        mesh=_sc_mesh(),
        compiler_params=pltpu.CompilerParams(needs_layout_passes=False),
        scratch_types=(
            [pltpu.VMEM((K,), jnp.int32) for _ in range(NIDX)]
            + [pltpu.VMEM((K,), jnp.int32) for _ in range(NIDX)]
            + [pltpu.VMEM((K,), jnp.float32) for _ in range(NIDX)]
            + [pltpu.VMEM((K, d), jnp.float32) for _ in range(NBUF)]
            + [pltpu.SemaphoreType.DMA for _ in range(NIDX + 2 * NBUF)]
            + [pltpu.VMEM_SHARED((n_pad, d), jnp.float32)]
        ),
    )
    def agg_kernel(row_hbm, col_hbm, ew_hbm, y_hbm, agg_hbm, *refs):
        rowb = refs[0:NIDX]
        colb = refs[NIDX:2 * NIDX]
        ewb = refs[2 * NIDX:3 * NIDX]
        rows = refs[3 * NIDX:3 * NIDX + NBUF]
        isem = refs[3 * NIDX + NBUF:3 * NIDX + NBUF + NIDX]
        gsem = refs[3 * NIDX + NBUF + NIDX:3 * NIDX + NBUF + NIDX + NBUF]
        ssem = refs[3 * NIDX + NBUF + NIDX + NBUF:
                    3 * NIDX + NBUF + NIDX + 2 * NBUF]
        acc = refs[3 * NIDX + NBUF + NIDX + 2 * NBUF]

        c_ax = lax.axis_index("c")
        s_ax = lax.axis_index("s")
        cpt = jnp.where(c_ax == 0, cpt0, cpt1)
        tile_base = jnp.where(
            c_ax == 0,
            s_ax * (cpt0 * K),
            N_SUBCORES * cpt0 * K + s_ax * (cpt1 * K),
        )

        @pl.loop(0, K)
        def _zero(i):
            for f in range(groups):
                rows[0][i, pl.ds(f * LANES, LANES)] = jnp.zeros(
                    (LANES,), jnp.float32)

        @pl.loop(0, rows_per_tile // K)
        def _zacc(t):
            pltpu.sync_copy(rows[0], acc.at[pl.ds(s_ax * rows_per_tile + t * K, K)])

        plsc.subcore_barrier()

        def issue_idx(ch, b):
            base = tile_base + ch * K
            pltpu.async_copy(row_hbm.at[pl.ds(base, K)], rowb[b], isem[b])
            pltpu.async_copy(col_hbm.at[pl.ds(base, K)], colb[b], isem[b])
            pltpu.async_copy(ew_hbm.at[pl.ds(base, K)], ewb[b], isem[b])

        def wait_idx(b):
            pltpu.make_async_copy(row_hbm.at[pl.ds(0, K)], rowb[b], isem[b]).wait()
            pltpu.make_async_copy(col_hbm.at[pl.ds(0, K)], colb[b], isem[b]).wait()
            pltpu.make_async_copy(ew_hbm.at[pl.ds(0, K)], ewb[b], isem[b]).wait()

        def issue_gather(b8, b4):
            pltpu.async_copy(y_hbm.at[rowb[b8]], rows[b4], gsem[b4])

        def wait_gather(b4):
            pltpu.make_async_copy(
                y_hbm.at[rowb[0]], rows[b4], gsem[b4]).wait()

        def wait_scat(b8, b4):
            pltpu.make_async_copy(rows[b4], acc.at[colb[b8]], ssem[b4]).wait()

        for ch in range(4):
            issue_idx(ch, ch)
        for ch in range(2):
            wait_idx(ch)
            issue_gather(ch, ch)

        @pl.loop(0, cpt, step=NIDX)
        def _main(g):
            for b in range(NIDX):
                ch = g + b
                b4 = b % NBUF
                tg8, tg4 = (b + 2) % NIDX, (b + 2) % NBUF
                ti = (b + 4) % NIDX

                wait_gather(b4)

                @pl.when(ch + 2 < cpt)
                def _prep_gather():
                    @pl.when(ch >= 2)
                    def _drain():
                        wait_scat(tg8, tg4)
                    wait_idx(tg8)
                    issue_gather(tg8, tg4)

                @pl.when(ch + 4 < cpt)
                def _prep_idx():
                    issue_idx(ch + 4, ti)

                @pl.loop(0, K, unroll=2)
                def _scale(j):
                    jv = jnp.broadcast_to(j, (LANES,)).astype(jnp.int32)
                    sv = plsc.load_gather(ewb[b], [jv])
                    for f in range(groups):
                        rows[b4][j, pl.ds(f * LANES, LANES)] = (
                            rows[b4][j, pl.ds(f * LANES, LANES)] * sv
                        )

                pltpu.async_copy(rows[b4], acc.at[colb[b]], ssem[b4], add=True)

        for b in range(NBUF):
            wait_scat(NBUF + b, b)

        plsc.subcore_barrier()
        pltpu.sync_copy(
            acc.at[pl.ds(s_ax * rows_per_tile, rows_per_tile)],
            agg_hbm.at[c_ax, pl.ds(s_ax * rows_per_tile, rows_per_tile)],
        )

    return agg_kernel


def _tc_xw_dis(x_p, W, deg3, n_pad, d):
    nb = n_pad // BR
    n_in = x_p.shape[0]

    def body(xb, wb, degb, yb, disb):
        dcol = degb[0] + degb[1] + 1.0
        dis = jnp.where(dcol > 0, lax.rsqrt(dcol), 0.0)
        xw = jnp.dot(xb[...], wb[...], preferred_element_type=jnp.float32)
        yb[...] = xw * dis
        disb[...] = dis

    return pl.pallas_call(
        body,
        grid=(nb,),
        in_specs=[
            pl.BlockSpec((BR, d), lambda i: (i, 0)),
            pl.BlockSpec((d, d), lambda i: (0, 0)),
            pl.BlockSpec((N_CORES, BR, 1), lambda i: (0, i, 0)),
        ],
        out_specs=[
            pl.BlockSpec((BR, d), lambda i: (i, 0)),
            pl.BlockSpec((BR, 1), lambda i: (i, 0)),
        ],
        out_shape=[
            jax.ShapeDtypeStruct((n_pad, d), jnp.float32),
            jax.ShapeDtypeStruct((n_pad, 1), jnp.float32),
        ],
    )(x_p, W, deg3)


def _tc_final(agg, y, dis, b2, n, n_pad, d):
    nb = n_pad // BR

    def body(aggb, yb, disb, bb, ob):
        s = (aggb[0] + aggb[1] + yb[...]) * disb[...] + bb[...]
        ob[...] = jnp.maximum(s, 0.0)

    return pl.pallas_call(
        body,
        grid=(nb,),
        in_specs=[
            pl.BlockSpec((N_CORES, BR, d), lambda i: (0, i, 0)),
            pl.BlockSpec((BR, d), lambda i: (i, 0)),
            pl.BlockSpec((BR, 1), lambda i: (i, 0)),
            pl.BlockSpec((1, d), lambda i: (0, 0)),
        ],
        out_specs=pl.BlockSpec((BR, d), lambda i: (i, 0)),
        out_shape=jax.ShapeDtypeStruct((n, d), jnp.float32),
    )(agg, y, dis, b2)


def kernel(x, edge_index, edge_weight, W, b):
    n, d = x.shape
    e = edge_index.shape[1]

    n_pad = ((n + NW * LANES - 1) // (NW * LANES)) * (NW * LANES)
    step = NW * K * NIDX
    e_pad = ((e + step - 1) // step) * step

    row = edge_index[0].astype(jnp.int32)
    col = edge_index[1].astype(jnp.int32)
    ew = edge_weight.astype(jnp.float32)
    if e_pad != e:
        pad = e_pad - e
        spread = (jnp.arange(pad, dtype=jnp.int32) * LANES) % n
        row = jnp.concatenate([row, spread])
        col = jnp.concatenate([col, spread])
        ew = jnp.concatenate([ew, jnp.zeros((pad,), jnp.float32)])
    deg_p = _make_deg_kernel(e_pad, n_pad)(col, ew)
    deg3 = deg_p.reshape(N_CORES, n_pad, 1)
    y, dis = _tc_xw_dis(x, W, deg3, n_pad, d)
    agg = _make_agg_kernel(e_pad, n_pad, d)(row, col, ew, y)
    return _tc_final(agg, y, dis, b.reshape(1, d), n, n_pad, d)


# speedup vs baseline: 41.0669x; 1.0372x over previous
import functools

import jax
import jax.numpy as jnp
from jax import lax
from jax.experimental import pallas as pl
from jax.experimental.pallas import tpu as pltpu
from jax.experimental.pallas import tpu_sc as plsc

N_CORES = 2
N_SUBCORES = 16
NW = N_CORES * N_SUBCORES
LANES = 16
K = 64
KD = 128
NBUF = 4
NIDX = 8
BR = 1024


def _sc_mesh():
    return plsc.VectorSubcoreMesh(core_axis_name="c", subcore_axis_name="s")


def _make_deg_kernel(e_pad, n_pad):
    cpt = e_pad // (NW * KD)
    rows_per_tile = n_pad // N_SUBCORES

    @functools.partial(
        pl.kernel,
        out_type=jax.ShapeDtypeStruct((N_CORES, n_pad), jnp.float32),
        mesh=_sc_mesh(),
        compiler_params=pltpu.CompilerParams(needs_layout_passes=False),
        scratch_types=(
            [pltpu.VMEM((KD,), jnp.int32) for _ in range(NBUF)]
            + [pltpu.VMEM((KD,), jnp.float32) for _ in range(NBUF)]
            + [pltpu.VMEM((rows_per_tile,), jnp.float32)]
            + [pltpu.SemaphoreType.DMA for _ in range(2 * NBUF)]
            + [pltpu.VMEM_SHARED((n_pad,), jnp.float32)]
        ),
    )
    def deg_kernel(col_hbm, ew_hbm, deg_hbm, *refs):
        colb = refs[0:NBUF]
        ewb = refs[NBUF:2 * NBUF]
        zbuf = refs[2 * NBUF]
        isem = refs[2 * NBUF + 1:2 * NBUF + 1 + NBUF]
        ssem = refs[2 * NBUF + 1 + NBUF:2 * NBUF + 1 + 2 * NBUF]
        acc = refs[2 * NBUF + 1 + 2 * NBUF]

        c_ax = lax.axis_index("c")
        s_ax = lax.axis_index("s")
        wid = c_ax * N_SUBCORES + s_ax
        tile_base = wid * cpt * KD

        @pl.loop(0, rows_per_tile // LANES)
        def _zero(i):
            zbuf[pl.ds(i * LANES, LANES)] = jnp.zeros((LANES,), jnp.float32)

        pltpu.sync_copy(zbuf, acc.at[pl.ds(s_ax * rows_per_tile, rows_per_tile)])
        plsc.subcore_barrier()

        def issue_idx(ch, b):
            base = tile_base + ch * KD
            pltpu.async_copy(col_hbm.at[pl.ds(base, KD)], colb[b], isem[b])
            pltpu.async_copy(ew_hbm.at[pl.ds(base, KD)], ewb[b], isem[b])

        def wait_idx(b):
            pltpu.make_async_copy(col_hbm.at[pl.ds(0, KD)], colb[b], isem[b]).wait()
            pltpu.make_async_copy(ew_hbm.at[pl.ds(0, KD)], ewb[b], isem[b]).wait()

        def wait_scat(b):
            pltpu.make_async_copy(ewb[b], acc.at[colb[b]], ssem[b]).wait()

        issue_idx(0, 0)
        issue_idx(1, 1)

        @pl.loop(0, cpt, step=NBUF)
        def _main(g):
            for b in range(NBUF):
                ch = g + b
                t = (b + 2) % NBUF

                @pl.when(ch + 2 < cpt)
                def _prep():
                    @pl.when(ch >= 2)
                    def _drain():
                        wait_scat(t)
                    issue_idx(ch + 2, t)

                wait_idx(b)
                pltpu.async_copy(ewb[b], acc.at[colb[b]], ssem[b], add=True)

        for b in range(NBUF):
            wait_scat(b)

        plsc.subcore_barrier()
        pltpu.sync_copy(
            acc.at[pl.ds(s_ax * rows_per_tile, rows_per_tile)],
            deg_hbm.at[c_ax, pl.ds(s_ax * rows_per_tile, rows_per_tile)],
        )

    return deg_kernel


def _make_agg_kernel(e_pad, n_pad, d):
    total_pt = e_pad // (N_SUBCORES * K)
    cpt0 = (total_pt // 2) // NIDX * NIDX
    cpt1 = total_pt - cpt0
    assert cpt1 % NIDX == 0 and cpt1 > 0
    rows_per_tile = n_pad // N_SUBCORES
    groups = d // LANES

    @functools.partial(
        pl.kernel,
        out_type=jax.ShapeDtypeStruct((N_CORES, n_pad, d), jnp.float32),
        mesh=_sc_mesh(),
        compiler_params=pltpu.CompilerParams(needs_layout_passes=False),
        scratch_types=(
            [pltpu.VMEM((K,), jnp.int32) for _ in range(NIDX)]
            + [pltpu.VMEM((K,), jnp.int32) for _ in range(NIDX)]
            + [pltpu.VMEM((K,), jnp.float32) for _ in range(NIDX)]
            + [pltpu.VMEM((K, d), jnp.float32) for _ in range(NBUF)]
            + [pltpu.SemaphoreType.DMA for _ in range(NIDX + 2 * NBUF)]
            + [pltpu.VMEM_SHARED((n_pad, d), jnp.float32)]
        ),
    )
    def agg_kernel(row_hbm, col_hbm, ew_hbm, y_hbm, agg_hbm, *refs):
        rowb = refs[0:NIDX]
        colb = refs[NIDX:2 * NIDX]
        ewb = refs[2 * NIDX:3 * NIDX]
        rows = refs[3 * NIDX:3 * NIDX + NBUF]
        isem = refs[3 * NIDX + NBUF:3 * NIDX + NBUF + NIDX]
        gsem = refs[3 * NIDX + NBUF + NIDX:3 * NIDX + NBUF + NIDX + NBUF]
        ssem = refs[3 * NIDX + NBUF + NIDX + NBUF:
                    3 * NIDX + NBUF + NIDX + 2 * NBUF]
        acc = refs[3 * NIDX + NBUF + NIDX + 2 * NBUF]

        c_ax = lax.axis_index("c")
        s_ax = lax.axis_index("s")
        cpt = jnp.where(c_ax == 0, cpt0, cpt1)
        tile_base = jnp.where(
            c_ax == 0,
            s_ax * (cpt0 * K),
            N_SUBCORES * cpt0 * K + s_ax * (cpt1 * K),
        )

        @pl.loop(0, K)
        def _zero(i):
            for f in range(groups):
                rows[0][i, pl.ds(f * LANES, LANES)] = jnp.zeros(
                    (LANES,), jnp.float32)

        @pl.loop(0, rows_per_tile // K)
        def _zacc(t):
            pltpu.sync_copy(rows[0], acc.at[pl.ds(s_ax * rows_per_tile + t * K, K)])

        plsc.subcore_barrier()

        def issue_idx(ch, b):
            base = tile_base + ch * K
            pltpu.async_copy(row_hbm.at[pl.ds(base, K)], rowb[b], isem[b])
            pltpu.async_copy(col_hbm.at[pl.ds(base, K)], colb[b], isem[b])
            pltpu.async_copy(ew_hbm.at[pl.ds(base, K)], ewb[b], isem[b])

        def wait_idx(b):
            pltpu.make_async_copy(row_hbm.at[pl.ds(0, K)], rowb[b], isem[b]).wait()
            pltpu.make_async_copy(col_hbm.at[pl.ds(0, K)], colb[b], isem[b]).wait()
            pltpu.make_async_copy(ew_hbm.at[pl.ds(0, K)], ewb[b], isem[b]).wait()

        def issue_gather(b8, b4):
            pltpu.async_copy(y_hbm.at[rowb[b8]], rows[b4], gsem[b4])

        def wait_gather(b4):
            pltpu.make_async_copy(
                y_hbm.at[rowb[0]], rows[b4], gsem[b4]).wait()

        def wait_scat(b8, b4):
            pltpu.make_async_copy(rows[b4], acc.at[colb[b8]], ssem[b4]).wait()

        for ch in range(4):
            issue_idx(ch, ch)
        for ch in range(2):
            wait_idx(ch)
            issue_gather(ch, ch)

        @pl.loop(0, cpt, step=NIDX)
        def _main(g):
            for b in range(NIDX):
                ch = g + b
                b4 = b % NBUF
                tg8, tg4 = (b + 2) % NIDX, (b + 2) % NBUF
                ti = (b + 4) % NIDX

                wait_gather(b4)

                @pl.when(ch + 2 < cpt)
                def _prep_gather():
                    @pl.when(ch >= 2)
                    def _drain():
                        wait_scat(tg8, tg4)
                    wait_idx(tg8)
                    issue_gather(tg8, tg4)

                @pl.when(ch + 4 < cpt)
                def _prep_idx():
                    issue_idx(ch + 4, ti)

                @pl.loop(0, K, unroll=2)
                def _scale(j):
                    jv = jnp.broadcast_to(j, (LANES,)).astype(jnp.int32)
                    sv = plsc.load_gather(ewb[b], [jv])
                    for f in range(groups):
                        rows[b4][j, pl.ds(f * LANES, LANES)] = (
                            rows[b4][j, pl.ds(f * LANES, LANES)] * sv
                        )

                pltpu.async_copy(rows[b4], acc.at[colb[b]], ssem[b4], add=True)

        for b in range(NBUF):
            wait_scat(NBUF + b, b)

        plsc.subcore_barrier()
        pltpu.sync_copy(
            acc.at[pl.ds(s_ax * rows_per_tile, rows_per_tile)],
            agg_hbm.at[c_ax, pl.ds(s_ax * rows_per_tile, rows_per_tile)],
        )

    return agg_kernel


def _tc_mm(x_p, W, n_pad, d):
    nb = n_pad // BR

    def body(xb, wb, ob):
        ob[...] = jnp.dot(xb[...], wb[...], preferred_element_type=jnp.float32)

    return pl.pallas_call(
        body,
        grid=(nb,),
        in_specs=[
            pl.BlockSpec((BR, d), lambda i: (i, 0)),
            pl.BlockSpec((d, d), lambda i: (0, 0)),
        ],
        out_specs=pl.BlockSpec((BR, d), lambda i: (i, 0)),
        out_shape=jax.ShapeDtypeStruct((n_pad, d), jnp.float32),
    )(x_p, W)


def _tc_scale(xw, deg_p, n_pad, d):
    nb = n_pad // BR

    def body(xwb, degb, yb, disb):
        dlane = degb[0] + degb[1] + 1.0
        dis = jnp.where(dlane > 0, lax.rsqrt(dlane), 0.0)
        dcol = dis.reshape(BR, 1)
        yb[...] = xwb[...] * dcol
        disb[...] = dcol

    return pl.pallas_call(
        body,
        grid=(nb,),
        in_specs=[
            pl.BlockSpec((BR, d), lambda i: (i, 0)),
            pl.BlockSpec((N_CORES, BR), lambda i: (0, i)),
        ],
        out_specs=[
            pl.BlockSpec((BR, d), lambda i: (i, 0)),
            pl.BlockSpec((BR, 1), lambda i: (i, 0)),
        ],
        out_shape=[
            jax.ShapeDtypeStruct((n_pad, d), jnp.float32),
            jax.ShapeDtypeStruct((n_pad, 1), jnp.float32),
        ],
    )(xw, deg_p)


def _tc_final(agg, y, dis, b2, n, n_pad, d):
    nb = n_pad // BR

    def body(aggb, yb, disb, bb, ob):
        s = (aggb[0] + aggb[1] + yb[...]) * disb[...] + bb[...]
        ob[...] = jnp.maximum(s, 0.0)

    return pl.pallas_call(
        body,
        grid=(nb,),
        in_specs=[
            pl.BlockSpec((N_CORES, BR, d), lambda i: (0, i, 0)),
            pl.BlockSpec((BR, d), lambda i: (i, 0)),
            pl.BlockSpec((BR, 1), lambda i: (i, 0)),
            pl.BlockSpec((1, d), lambda i: (0, 0)),
        ],
        out_specs=pl.BlockSpec((BR, d), lambda i: (i, 0)),
        out_shape=jax.ShapeDtypeStruct((n, d), jnp.float32),
    )(agg, y, dis, b2)


def kernel(x, edge_index, edge_weight, W, b):
    n, d = x.shape
    e = edge_index.shape[1]

    n_pad = ((n + NW * LANES - 1) // (NW * LANES)) * (NW * LANES)
    step = NW * K * NIDX
    e_pad = ((e + step - 1) // step) * step

    row = edge_index[0].astype(jnp.int32)
    col = edge_index[1].astype(jnp.int32)
    ew = edge_weight.astype(jnp.float32)
    if e_pad != e:
        pad = e_pad - e
        spread = (jnp.arange(pad, dtype=jnp.int32) * LANES) % n
        row = jnp.concatenate([row, spread])
        col = jnp.concatenate([col, spread])
        ew = jnp.concatenate([ew, jnp.zeros((pad,), jnp.float32)])
    deg_p = _make_deg_kernel(e_pad, n_pad)(col, ew)
    xw = _tc_mm(x, W, n_pad, d)
    y, dis = _tc_scale(xw, deg_p, n_pad, d)
    agg = _make_agg_kernel(e_pad, n_pad, d)(row, col, ew, y)
    return _tc_final(agg, y, dis, b.reshape(1, d), n, n_pad, d)
